# Initial kernel scaffold; baseline (speedup 1.0000x reference)
import jax
import jax.numpy as jnp
from jax.experimental import pallas as pl


def kernel(x, Wg, bg, Wn, bn, W1, b1, W2, b2):
    raise NotImplementedError("write your pallas kernel here")


# speedup vs baseline: 1.0007x; 1.0007x over previous
import functools

import jax
import jax.numpy as jnp
from jax.experimental import pallas as pl
from jax.experimental.pallas import tpu as pltpu

T = 4096
D = 768
DFF = 3072
E = 8
TOPK = 2

BT = 512


def _moe_block(x_ref, g_ref, w1_ref, b1_ref, w2_ref, b2_ref, out_ref):
    e = pl.program_id(1)

    @pl.when(e == 0)
    def _():
        out_ref[...] = jnp.zeros_like(out_ref)

    x = x_ref[...]
    h = jnp.maximum(
        jnp.dot(x, w1_ref[0], preferred_element_type=jnp.float32) + b1_ref[0],
        0.0,
    )
    y = jnp.dot(h, w2_ref[0], preferred_element_type=jnp.float32) + b2_ref[0]
    lane = jax.lax.broadcasted_iota(jnp.int32, (BT, E), 1)
    g = jnp.sum(jnp.where(lane == e, g_ref[...], 0.0), axis=1, keepdims=True)
    out_ref[...] += g * y


def kernel(x, Wg, bg, Wn, bn, W1, b1, W2, b2):
    logits = x @ Wg + bg
    noise_logits = x @ Wn + bn
    base_noise = jax.random.normal(jax.random.key(42), logits.shape, dtype=logits.dtype)
    noisy = logits + base_noise * jax.nn.softplus(noise_logits)
    topk_logits, topk_idx = jax.lax.top_k(noisy, TOPK)
    sel = jax.nn.one_hot(topk_idx, E, dtype=noisy.dtype).sum(axis=-2) > 0
    sparse_logits = jnp.where(sel, noisy, -jnp.inf)
    gating = jax.nn.softmax(sparse_logits, axis=-1)

    grid = (T // BT, E)
    out = pl.pallas_call(
        _moe_block,
        grid=grid,
        in_specs=[
            pl.BlockSpec((BT, D), lambda t, e: (t, 0)),
            pl.BlockSpec((BT, E), lambda t, e: (t, 0)),
            pl.BlockSpec((1, D, DFF), lambda t, e: (e, 0, 0)),
            pl.BlockSpec((1, 1, DFF), lambda t, e: (e, 0, 0)),
            pl.BlockSpec((1, DFF, D), lambda t, e: (e, 0, 0)),
            pl.BlockSpec((1, 1, D), lambda t, e: (e, 0, 0)),
        ],
        out_specs=pl.BlockSpec((BT, D), lambda t, e: (t, 0)),
        out_shape=jax.ShapeDtypeStruct((T, D), jnp.float32),
    )(x, gating, W1, b1[:, None, :], W2, b2[:, None, :])
    return out
